# Initial kernel scaffold; baseline (speedup 1.0000x reference)
import jax
import jax.numpy as jnp
from jax.experimental import pallas as pl


def kernel(x_author, x_paper, params, ei_writes, ei_written):
    raise NotImplementedError("write your pallas kernel here")


# speedup vs baseline: 3.6141x; 3.6141x over previous
import functools

import jax
import jax.numpy as jnp
from jax import lax
from jax.experimental import pallas as pl
from jax.experimental.pallas import tpu as pltpu
from jax.experimental.pallas import tpu_sc as plsc

_N = 10000
_D = 128
_E = 320000

_NC = 2
_NS = 16
_NW = _NC * _NS
_CH = 128
_EPW = 10112
_EP = _EPW * _NW
_NCHUNK = _EPW // _CH
_ROWS = 10112
_RPS = _ROWS // _NS


def _segsum_body(h_hbm, src_hbm, dst_hbm, out_hbm, acc, rows, idx_s, idx_d, sem):
    c = lax.axis_index("c")
    s = lax.axis_index("s")
    wid = s * _NC + c

    def _zrow(r, _):
        def _zcol(k, __):
            rows[r, pl.ds(k * 16, 16)] = jnp.zeros((16,), jnp.float32)
            return 0
        return lax.fori_loop(0, _D // 16, _zcol, 0)
    lax.fori_loop(0, _CH, _zrow, 0)
    for k in range(_RPS // _CH):
        pltpu.sync_copy(rows, acc.at[pl.ds(s * _RPS + k * _CH, _CH)])
    _rem = _RPS % _CH
    pltpu.sync_copy(rows.at[pl.ds(0, _rem)],
                    acc.at[pl.ds(s * _RPS + (_RPS // _CH) * _CH, _rem)])
    plsc.subcore_barrier()

    def _chunk(j, _):
        off = wid * _EPW + j * _CH
        pltpu.sync_copy(src_hbm.at[pl.ds(off, _CH)], idx_s)
        pltpu.sync_copy(dst_hbm.at[pl.ds(off, _CH)], idx_d)
        pltpu.async_copy(h_hbm.at[idx_s], rows, sem).wait()
        pltpu.sync_copy(rows, acc.at[idx_d], add=True)
        return 0
    lax.fori_loop(0, _NCHUNK, _chunk, 0)
    plsc.subcore_barrier()

    for k in range(_RPS // _CH):
        r0 = s * _RPS + k * _CH
        pltpu.sync_copy(acc.at[pl.ds(r0, _CH)], rows)
        pltpu.sync_copy(rows, out_hbm.at[c, pl.ds(r0, _CH)])
    r0 = s * _RPS + (_RPS // _CH) * _CH
    pltpu.sync_copy(acc.at[pl.ds(r0, _rem)], rows.at[pl.ds(0, _rem)])
    pltpu.sync_copy(rows.at[pl.ds(0, _rem)], out_hbm.at[c, pl.ds(r0, _rem)])


def _segsum(h, src, dst):
    mesh = plsc.VectorSubcoreMesh(core_axis_name="c", subcore_axis_name="s")
    f = pl.kernel(
        _segsum_body,
        mesh=mesh,
        out_type=jax.ShapeDtypeStruct((_NC, _ROWS, _D), jnp.float32),
        scratch_types=[
            pltpu.VMEM_SHARED((_ROWS, _D), jnp.float32),
            pltpu.VMEM((_CH, _D), jnp.float32),
            pltpu.VMEM((_CH,), jnp.int32),
            pltpu.VMEM((_CH,), jnp.int32),
            pltpu.SemaphoreType.DMA,
        ],
    )
    return f(h, src, dst)


def _pad_edges(ei):
    pad = _EP - _E
    src = jnp.concatenate([ei[0], jnp.zeros((pad,), jnp.int32)])
    dst = jnp.concatenate([ei[1], jnp.full((pad,), _N, jnp.int32)])
    return src, dst


def _matmul_t(x, w):
    return lax.dot_general(x, w, (((1,), (1,)), ((), ())),
                           preferred_element_type=jnp.float32)


def _lin_body(x_ref, w_ref, b_ref, o_ref):
    o_ref[...] = _matmul_t(x_ref[...], w_ref[...]) + b_ref[...]


def _lin(x, w, b):
    return pl.pallas_call(
        _lin_body,
        out_shape=jax.ShapeDtypeStruct((x.shape[0], w.shape[0]), jnp.float32),
    )(x, w, b.reshape(1, -1))


def _mlp_core(x_ref, a_ref, eps_ref, w1_ref, b1_ref, g_ref, be_ref, w2_ref, b2_ref):
    agg = a_ref[0, 0:_N, :] + a_ref[1, 0:_N, :]
    h = (1.0 + eps_ref[0]) * x_ref[...] + agg
    t = _matmul_t(h, w1_ref[...]) + b1_ref[...]
    mean = jnp.mean(t, axis=0, keepdims=True)
    var = jnp.mean((t - mean) ** 2, axis=0, keepdims=True)
    t = (t - mean) * lax.rsqrt(var + 1e-5) * g_ref[...] + be_ref[...]
    t = jnp.maximum(t, 0.0)
    t = _matmul_t(t, w2_ref[...]) + b2_ref[...]
    return jnp.maximum(t, 0.0)


def _gin_mlp_body(x_ref, a_ref, eps_ref, w1_ref, b1_ref, g_ref, be_ref,
                  w2_ref, b2_ref, o_ref):
    o_ref[...] = _mlp_core(x_ref, a_ref, eps_ref, w1_ref, b1_ref, g_ref,
                           be_ref, w2_ref, b2_ref)


def _gin_mlp_final_body(x_ref, a_ref, eps_ref, w1_ref, b1_ref, g_ref, be_ref,
                        w2_ref, b2_ref, wf_ref, bf_ref, o_ref):
    t = _mlp_core(x_ref, a_ref, eps_ref, w1_ref, b1_ref, g_ref,
                  be_ref, w2_ref, b2_ref)
    o_ref[...] = _matmul_t(t, wf_ref[...]) + bf_ref[...]


def _mlp_args(x, agg, p):
    return (x, agg, p["eps"].reshape(1),
            p["W1"], p["b1"].reshape(1, -1),
            p["gamma"].reshape(1, -1), p["beta"].reshape(1, -1),
            p["W2"], p["b2"].reshape(1, -1))


_SMEM1 = pl.BlockSpec(memory_space=pltpu.SMEM)


def _gin_mlp(x, agg, p):
    specs = [None, None, _SMEM1] + [None] * 6
    specs = [s if s is not None else pl.BlockSpec() for s in specs]
    return pl.pallas_call(
        _gin_mlp_body,
        in_specs=specs,
        out_shape=jax.ShapeDtypeStruct((_N, _D), jnp.float32),
    )(*_mlp_args(x, agg, p))


def _gin_mlp_final(x, agg, p, pf):
    specs = [None, None, _SMEM1] + [None] * 8
    specs = [s if s is not None else pl.BlockSpec() for s in specs]
    return pl.pallas_call(
        _gin_mlp_final_body,
        in_specs=specs,
        out_shape=jax.ShapeDtypeStruct((_N, pf["W"].shape[0]), jnp.float32),
    )(*_mlp_args(x, agg, p), pf["W"], pf["b"].reshape(1, -1))


def kernel(x_author, x_paper, params, ei_writes, ei_written):
    p = params
    src_w, dst_w = _pad_edges(ei_writes)
    src_n, dst_n = _pad_edges(ei_written)

    h_a = _lin(x_author, p["lin_author"]["W"], p["lin_author"]["b"])
    h_p = _lin(x_paper, p["lin_paper"]["W"], p["lin_paper"]["b"])

    l1, l2 = p["layers"][0], p["layers"][1]
    agg_p = _segsum(h_a, src_w, dst_w)
    agg_a = _segsum(h_p, src_n, dst_n)
    h_p1 = _gin_mlp(h_p, agg_p, l1["writes"])
    h_a1 = _gin_mlp(h_a, agg_a, l1["written"])

    agg_a2 = _segsum(h_p1, src_n, dst_n)
    return _gin_mlp_final(h_a1, agg_a2, l2["written"], p["final"])
